# baseline (speedup 1.0000x reference)
import functools
import math

import jax
import jax.numpy as jnp
from jax.experimental import pallas as pl
from jax.experimental.pallas import tpu as pltpu

_SPARSE_RATIO = 0.6


def _normalize(s):
    s_min = jnp.min(s, axis=-1, keepdims=True)
    s_max = jnp.max(s, axis=-1, keepdims=True)
    return (s - s_min) / (s_max - s_min + 1e-08)


def _row_mask(sa, m2, m3, w, k):
    score = w[0, 0] * _normalize(sa) + w[0, 1] * _normalize(m2) \
        + w[0, 2] * _normalize(m3)

    bits = jax.lax.bitcast_convert_type(score, jnp.int32)
    r = score.shape[0]

    def bits_step(_, carry):
        lo, hi = carry
        mid = lo + jax.lax.div(hi - lo, 2)
        cnt = jnp.sum((bits > mid).astype(jnp.int32), axis=-1, keepdims=True)
        ge = cnt >= k
        lo = jnp.where(ge, mid, lo)
        hi = jnp.where(ge, hi, mid)
        return lo, hi

    lo0 = jnp.full((r, 1), -1, dtype=jnp.int32)
    hi0 = jnp.full((r, 1), 0x7F800000, dtype=jnp.int32)
    _, t_bits = jax.lax.fori_loop(0, 32, bits_step, (lo0, hi0))

    gt = bits > t_bits
    eq = bits == t_bits
    count_gt = jnp.sum(gt.astype(jnp.int32), axis=-1, keepdims=True)
    need = k - count_gt

    idx = jax.lax.broadcasted_iota(jnp.int32, score.shape, 1)

    def idx_step(_, carry):
        lo, hi = carry
        mid = lo + jax.lax.div(hi - lo, 2)
        cnt = jnp.sum((eq & (idx < mid)).astype(jnp.int32), axis=-1, keepdims=True)
        ge = cnt >= need
        lo = jnp.where(ge, lo, mid)
        hi = jnp.where(ge, mid, hi)
        return lo, hi

    lo0 = jnp.zeros((r, 1), dtype=jnp.int32)
    hi0 = jnp.full((r, 1), score.shape[1], dtype=jnp.int32)
    _, m = jax.lax.fori_loop(0, 14, idx_step, (lo0, hi0))

    return (gt | (eq & (idx < m))).astype(jnp.float32)


def _fused_kernel(sa_ref, m2_ref, m3_ref, w_ref, tok_ref,
                  out_ref, mask_out_ref, mask_scr, *, k, nb):
    i = pl.program_id(0)
    j = pl.program_id(1)

    @pl.when(j == 0)
    def _():
        mask_scr[...] = _row_mask(
            sa_ref[pl.ds(i, 1), :], m2_ref[pl.ds(i, 1), :],
            m3_ref[pl.ds(i, 1), :], w_ref[...], k)

    mask_out_ref[...] = mask_scr[...][None]
    out_ref[...] = tok_ref[...] * mask_scr[:, pl.ds(j * nb, nb)][..., None]


def kernel(tokens, self_attention, cross_attention_m2, cross_attention_m3, modal_weights):
    b, n, c = tokens.shape
    k = max(1, math.ceil(n * _SPARSE_RATIO))
    nb = 4096

    masked_tokens, mask3d = pl.pallas_call(
        functools.partial(_fused_kernel, k=k, nb=nb),
        grid=(b, n // nb),
        in_specs=[
            pl.BlockSpec((b, n), lambda i, j: (0, 0)),
            pl.BlockSpec((b, n), lambda i, j: (0, 0)),
            pl.BlockSpec((b, n), lambda i, j: (0, 0)),
            pl.BlockSpec((1, 3), lambda i, j: (0, 0)),
            pl.BlockSpec((1, nb, c), lambda i, j: (i, j, 0)),
        ],
        out_specs=[
            pl.BlockSpec((1, nb, c), lambda i, j: (i, j, 0)),
            pl.BlockSpec((1, 1, n), lambda i, j: (i, 0, 0)),
        ],
        out_shape=[
            jax.ShapeDtypeStruct((b, n, c), jnp.float32),
            jax.ShapeDtypeStruct((b, 1, n), jnp.float32),
        ],
        scratch_shapes=[pltpu.VMEM((1, n), jnp.float32)],
    )(self_attention, cross_attention_m2, cross_attention_m3,
      modal_weights.reshape(1, 3), tokens)

    return (masked_tokens, mask3d.reshape(b, n))


# speedup vs baseline: 1.0855x; 1.0855x over previous
import functools
import math

import jax
import jax.numpy as jnp
from jax.experimental import pallas as pl
from jax.experimental.pallas import tpu as pltpu

_SPARSE_RATIO = 0.6


def _normalize(s):
    s_min = jnp.min(s, axis=-1, keepdims=True)
    s_max = jnp.max(s, axis=-1, keepdims=True)
    return (s - s_min) / (s_max - s_min + 1e-08)


def _row_mask(sa, m2, m3, w, k):
    score = w[0, 0] * _normalize(sa) + w[0, 1] * _normalize(m2) \
        + w[0, 2] * _normalize(m3)

    bits = jax.lax.bitcast_convert_type(score, jnp.int32)
    r = score.shape[0]

    def bits_step(_, carry):
        lo, hi = carry
        mid = lo + jax.lax.div(hi - lo, 2)
        cnt = jnp.sum((bits > mid).astype(jnp.int32), axis=-1, keepdims=True)
        ge = cnt >= k
        lo = jnp.where(ge, mid, lo)
        hi = jnp.where(ge, hi, mid)
        return lo, hi

    lo0 = jnp.full((r, 1), -1, dtype=jnp.int32)
    hi0 = jnp.full((r, 1), 0x7F800000, dtype=jnp.int32)
    _, t_bits = jax.lax.fori_loop(0, 32, bits_step, (lo0, hi0))

    gt = bits > t_bits
    eq = bits == t_bits
    count_gt = jnp.sum(gt.astype(jnp.int32), axis=-1, keepdims=True)
    need = k - count_gt

    idx = jax.lax.broadcasted_iota(jnp.int32, score.shape, 1)

    def idx_step(_, carry):
        lo, hi = carry
        mid = lo + jax.lax.div(hi - lo, 2)
        cnt = jnp.sum((eq & (idx < mid)).astype(jnp.int32), axis=-1, keepdims=True)
        ge = cnt >= need
        lo = jnp.where(ge, lo, mid)
        hi = jnp.where(ge, mid, hi)
        return lo, hi

    lo0 = jnp.zeros((r, 1), dtype=jnp.int32)
    hi0 = jnp.full((r, 1), score.shape[1], dtype=jnp.int32)
    _, m = jax.lax.fori_loop(0, 14, idx_step, (lo0, hi0))

    return (gt | (eq & (idx < m))).astype(jnp.float32)


def _fused_kernel(sa_ref, m2_ref, m3_ref, w_ref, tok_ref,
                  out_ref, mask_out_ref, mask_scr, *, k, nb):
    i = pl.program_id(0)
    j = pl.program_id(1)

    @pl.when((i == 0) & (j == 0))
    def _():
        mask_scr[...] = _row_mask(
            sa_ref[...], m2_ref[...], m3_ref[...], w_ref[...], k)

    mask_out_ref[...] = mask_scr[pl.ds(i, 1), :][None]
    out_ref[...] = tok_ref[...] * mask_scr[pl.ds(i, 1), pl.ds(j * nb, nb)][..., None]


def kernel(tokens, self_attention, cross_attention_m2, cross_attention_m3, modal_weights):
    b, n, c = tokens.shape
    k = max(1, math.ceil(n * _SPARSE_RATIO))
    nb = 4096

    masked_tokens, mask3d = pl.pallas_call(
        functools.partial(_fused_kernel, k=k, nb=nb),
        grid=(b, n // nb),
        in_specs=[
            pl.BlockSpec((b, n), lambda i, j: (0, 0)),
            pl.BlockSpec((b, n), lambda i, j: (0, 0)),
            pl.BlockSpec((b, n), lambda i, j: (0, 0)),
            pl.BlockSpec((1, 3), lambda i, j: (0, 0)),
            pl.BlockSpec((1, nb, c), lambda i, j: (i, j, 0)),
        ],
        out_specs=[
            pl.BlockSpec((1, nb, c), lambda i, j: (i, j, 0)),
            pl.BlockSpec((1, 1, n), lambda i, j: (i, 0, 0)),
        ],
        out_shape=[
            jax.ShapeDtypeStruct((b, n, c), jnp.float32),
            jax.ShapeDtypeStruct((b, 1, n), jnp.float32),
        ],
        scratch_shapes=[pltpu.VMEM((b, n), jnp.float32)],
    )(self_attention, cross_attention_m2, cross_attention_m3,
      modal_weights.reshape(1, 3), tokens)

    return (masked_tokens, mask3d.reshape(b, n))
